# baseline (speedup 1.0000x reference)
import math

import jax
import jax.numpy as jnp
from jax import lax
from jax.experimental import pallas as pl
from jax.experimental.pallas import tpu as pltpu
from jax.experimental.pallas import tpu_sc as plsc

_CUTOFF = 0.5
_E_PARENT = 6_400_000
_E_FILTER = 3_200_000

_K = 2560
_NCHUNKS = _E_FILTER // _K
_NW = 32
_JMAX = -(-_NCHUNKS // _NW)
_NSETS = 3
_JTRIPS = -(-_JMAX // _NSETS)

_S3 = -1.0 / 6.0
_S5 = 1.0 / 120.0
_S7 = -1.0 / 5040.0
_S9 = 1.0 / 362880.0


def _body(vx_hbm, vy_hbm, vz_hbm, dist_hbm, idx_hbm,
          ox_hbm, oy_hbm, oz_hbm, distf_hbm, sw_hbm, mask_hbm,
          idx_v0, px_v0, py_v0, pz_v0, dist_v0, sw_v0, mask_v0,
          idx_v1, px_v1, py_v1, pz_v1, dist_v1, sw_v1, mask_v1,
          idx_v2, px_v2, py_v2, pz_v2, dist_v2, sw_v2, mask_v2,
          sem_d0, sem_v0, sem_w0, sem_i0,
          sem_d1, sem_v1, sem_w1, sem_i1,
          sem_d2, sem_v2, sem_w2, sem_i2):
    wid = lax.axis_index("s") * 2 + lax.axis_index("c")
    ones_i = jnp.ones((16,), jnp.int32)
    zeros_i = jnp.zeros((16,), jnp.int32)

    sets = (
        (idx_v0, px_v0, py_v0, pz_v0, dist_v0, sw_v0, mask_v0,
         sem_d0, sem_v0, sem_w0, sem_i0),
        (idx_v1, px_v1, py_v1, pz_v1, dist_v1, sw_v1, mask_v1,
         sem_d1, sem_v1, sem_w1, sem_i1),
        (idx_v2, px_v2, py_v2, pz_v2, dist_v2, sw_v2, mask_v2,
         sem_d2, sem_v2, sem_w2, sem_i2),
    )

    def prefetch_idx(j, s):
        idx_v, *_rest, sem_i = s
        c = wid + _NW * j

        @pl.when(c < _NCHUNKS)
        def _():
            pltpu.async_copy(idx_hbm.at[pl.ds(c * _K, _K)], idx_v, sem_i)

    def drain_writes(s):
        (_idx, px_v, py_v, pz_v, dist_v, sw_v, mask_v,
         _d, _v, sem_w, _i) = s
        z = pl.ds(0, _K)
        pltpu.make_async_copy(dist_v, distf_hbm.at[z], sem_w).wait()
        pltpu.make_async_copy(sw_v, sw_hbm.at[z], sem_w).wait()
        pltpu.make_async_copy(mask_v, mask_hbm.at[z], sem_w).wait()
        pltpu.make_async_copy(px_v, ox_hbm.at[z], sem_w).wait()
        pltpu.make_async_copy(py_v, oy_hbm.at[z], sem_w).wait()
        pltpu.make_async_copy(pz_v, oz_hbm.at[z], sem_w).wait()

    def fire(j, s, drain):
        (idx_v, px_v, py_v, pz_v, dist_v, _sw, _mk,
         sem_d, sem_v, _w, sem_i) = s
        c = wid + _NW * j

        @pl.when(c < _NCHUNKS)
        def _():
            if drain:
                drain_writes(s)
            pltpu.make_async_copy(
                idx_hbm.at[pl.ds(c * _K, _K)], idx_v, sem_i).wait()
            pltpu.async_copy(dist_hbm.at[idx_v], dist_v, sem_d)
            pltpu.async_copy(vx_hbm.at[idx_v], px_v, sem_v)
            pltpu.async_copy(vy_hbm.at[idx_v], py_v, sem_v)
            pltpu.async_copy(vz_hbm.at[idx_v], pz_v, sem_v)

    def finish(j, s):
        (idx_v, px_v, py_v, pz_v, dist_v, sw_v, mask_v,
         sem_d, sem_v, sem_w, _i) = s
        c = wid + _NW * j

        @pl.when(c < _NCHUNKS)
        def _():
            base = c * _K
            pltpu.make_async_copy(dist_hbm.at[idx_v], dist_v, sem_d).wait()

            def group(g, carry):
                o = g * 16
                d = dist_v[pl.ds(o, 16)]
                m = d < _CUTOFF
                z = d * (math.pi / _CUTOFF) - (math.pi / 2.0)
                z2 = z * z
                s_ = z * (1.0 + z2 * (_S3 + z2 * (_S5 + z2 * (_S7 + z2 * _S9))))
                sw_v[pl.ds(o, 16)] = jnp.where(m, 0.5 - 0.5 * s_, 0.0)
                mask_v[pl.ds(o, 16)] = jnp.where(m, ones_i, zeros_i)
                return carry

            lax.fori_loop(0, _K // 16, group, 0, unroll=4)

            pltpu.async_copy(dist_v, distf_hbm.at[pl.ds(base, _K)], sem_w)
            pltpu.async_copy(sw_v, sw_hbm.at[pl.ds(base, _K)], sem_w)
            pltpu.async_copy(mask_v, mask_hbm.at[pl.ds(base, _K)], sem_w)
            pltpu.make_async_copy(vx_hbm.at[idx_v], px_v, sem_v).wait()
            pltpu.make_async_copy(vy_hbm.at[idx_v], py_v, sem_v).wait()
            pltpu.make_async_copy(vz_hbm.at[idx_v], pz_v, sem_v).wait()
            pltpu.async_copy(px_v, ox_hbm.at[pl.ds(base, _K)], sem_w)
            pltpu.async_copy(py_v, oy_hbm.at[pl.ds(base, _K)], sem_w)
            pltpu.async_copy(pz_v, oz_hbm.at[pl.ds(base, _K)], sem_w)
            prefetch_idx(j + _NSETS, s)

    for k in range(_NSETS):
        prefetch_idx(k, sets[k])
    for k in range(_NSETS):
        fire(k, sets[k], drain=False)

    def trip_body(jj, carry):
        j0 = _NSETS * jj
        for k in range(_NSETS):
            finish(j0 + k, sets[k])
            fire(j0 + k + _NSETS, sets[k], drain=True)
        return carry

    lax.fori_loop(0, _JTRIPS, trip_body, 0)
    for k in range(_NSETS):
        drain_writes(sets[k])


@jax.jit
def _run(vec, distances, filter_indices):
    mesh = plsc.VectorSubcoreMesh(core_axis_name="c", subcore_axis_name="s")
    f32 = jnp.float32
    set_scratch = [
        pltpu.VMEM((_K,), jnp.int32),
        pltpu.VMEM((_K,), f32),
        pltpu.VMEM((_K,), f32),
        pltpu.VMEM((_K,), f32),
        pltpu.VMEM((_K,), f32),
        pltpu.VMEM((_K,), f32),
        pltpu.VMEM((_K,), jnp.int32),
    ]
    sems = [pltpu.SemaphoreType.DMA] * (4 * _NSETS)
    fn = pl.kernel(
        _body,
        out_type=[
            jax.ShapeDtypeStruct((_E_FILTER,), f32),
            jax.ShapeDtypeStruct((_E_FILTER,), f32),
            jax.ShapeDtypeStruct((_E_FILTER,), f32),
            jax.ShapeDtypeStruct((_E_FILTER,), f32),
            jax.ShapeDtypeStruct((_E_FILTER,), f32),
            jax.ShapeDtypeStruct((_E_FILTER,), jnp.int32),
        ],
        mesh=mesh,
        scratch_types=set_scratch * _NSETS + sems,
    )
    ox, oy, oz, dist_f, switch, mask_i32 = fn(
        vec[:, 0], vec[:, 1], vec[:, 2], distances, filter_indices)
    vec_f = jnp.stack([ox, oy, oz], axis=1)
    return vec_f, dist_f, switch, mask_i32.astype(jnp.bool_)


def kernel(vec, distances, filter_indices):
    return _run(vec, distances, filter_indices)


# speedup vs baseline: 1.0072x; 1.0072x over previous
import math

import jax
import jax.numpy as jnp
from jax import lax
from jax.experimental import pallas as pl
from jax.experimental.pallas import tpu as pltpu
from jax.experimental.pallas import tpu_sc as plsc

_CUTOFF = 0.5
_E_PARENT = 6_400_000
_E_FILTER = 3_200_000

_K = 2560
_NCHUNKS = _E_FILTER // _K
_NW = 32
_JMAX = -(-_NCHUNKS // _NW)
_JPAIRS = (_JMAX + 1) // 2

_S3 = -1.0 / 6.0
_S5 = 1.0 / 120.0
_S7 = -1.0 / 5040.0
_S9 = 1.0 / 362880.0


def _sweep(prefetch_idx, fire, finish, drain_writes, sets):
    prefetch_idx(0, sets[0])
    prefetch_idx(1, sets[1])
    fire(0, sets[0], False)
    fire(1, sets[1], False)
    finish(0, sets[0])
    fire(2, sets[0], True)
    finish(1, sets[1])

    def pair_body(jj, carry):
        j0 = 2 * jj
        fire(j0 + 1, sets[1], True)
        finish(j0, sets[0])
        fire(j0 + 2, sets[0], True)
        finish(j0 + 1, sets[1])
        return carry

    lax.fori_loop(1, _JPAIRS, pair_body, 0)
    drain_writes(sets[0])
    drain_writes(sets[1])


def _dist_body(dist_hbm, idx_hbm,
               distf_hbm, sw_hbm, mask_hbm,
               idx_v0, dist_v0, sw_v0, mask_v0,
               idx_v1, dist_v1, sw_v1, mask_v1,
               sem_d0, sem_w0, sem_i0,
               sem_d1, sem_w1, sem_i1):
    wid = lax.axis_index("s") * 2 + lax.axis_index("c")
    ones_i = jnp.ones((16,), jnp.int32)
    zeros_i = jnp.zeros((16,), jnp.int32)

    sets = (
        (idx_v0, dist_v0, sw_v0, mask_v0, sem_d0, sem_w0, sem_i0),
        (idx_v1, dist_v1, sw_v1, mask_v1, sem_d1, sem_w1, sem_i1),
    )

    def prefetch_idx(j, s):
        idx_v, *_rest, sem_i = s
        c = wid + _NW * j

        @pl.when(c < _NCHUNKS)
        def _():
            pltpu.async_copy(idx_hbm.at[pl.ds(c * _K, _K)], idx_v, sem_i)

    def drain_writes(s):
        (_idx, dist_v, sw_v, mask_v, _d, sem_w, _i) = s
        z = pl.ds(0, _K)
        pltpu.make_async_copy(dist_v, distf_hbm.at[z], sem_w).wait()
        pltpu.make_async_copy(sw_v, sw_hbm.at[z], sem_w).wait()
        pltpu.make_async_copy(mask_v, mask_hbm.at[z], sem_w).wait()

    def fire(j, s, drain):
        (idx_v, dist_v, _sw, _mk, sem_d, _w, sem_i) = s
        c = wid + _NW * j

        @pl.when(c < _NCHUNKS)
        def _():
            if drain:
                drain_writes(s)
            pltpu.make_async_copy(
                idx_hbm.at[pl.ds(c * _K, _K)], idx_v, sem_i).wait()
            pltpu.async_copy(dist_hbm.at[idx_v], dist_v, sem_d)

    def finish(j, s):
        (idx_v, dist_v, sw_v, mask_v, sem_d, sem_w, _i) = s
        c = wid + _NW * j

        @pl.when(c < _NCHUNKS)
        def _():
            base = c * _K
            pltpu.make_async_copy(dist_hbm.at[idx_v], dist_v, sem_d).wait()
            prefetch_idx(j + 2, s)

            def group(g, carry):
                o = g * 16
                d = dist_v[pl.ds(o, 16)]
                m = d < _CUTOFF
                z = d * (math.pi / _CUTOFF) - (math.pi / 2.0)
                z2 = z * z
                s_ = z * (1.0 + z2 * (_S3 + z2 * (_S5 + z2 * (_S7 + z2 * _S9))))
                sw_v[pl.ds(o, 16)] = jnp.where(m, 0.5 - 0.5 * s_, 0.0)
                mask_v[pl.ds(o, 16)] = jnp.where(m, ones_i, zeros_i)
                return carry

            lax.fori_loop(0, _K // 16, group, 0, unroll=4)

            pltpu.async_copy(dist_v, distf_hbm.at[pl.ds(base, _K)], sem_w)
            pltpu.async_copy(sw_v, sw_hbm.at[pl.ds(base, _K)], sem_w)
            pltpu.async_copy(mask_v, mask_hbm.at[pl.ds(base, _K)], sem_w)

    _sweep(prefetch_idx, fire, finish, drain_writes, sets)


def _vec_body(vx_hbm, vy_hbm, vz_hbm, idx_hbm,
              ox_hbm, oy_hbm, oz_hbm,
              idx_v0, px_v0, py_v0, pz_v0,
              idx_v1, px_v1, py_v1, pz_v1,
              sem_v0, sem_w0, sem_i0,
              sem_v1, sem_w1, sem_i1):
    wid = lax.axis_index("s") * 2 + lax.axis_index("c")

    sets = (
        (idx_v0, px_v0, py_v0, pz_v0, sem_v0, sem_w0, sem_i0),
        (idx_v1, px_v1, py_v1, pz_v1, sem_v1, sem_w1, sem_i1),
    )

    def prefetch_idx(j, s):
        idx_v, *_rest, sem_i = s
        c = wid + _NW * j

        @pl.when(c < _NCHUNKS)
        def _():
            pltpu.async_copy(idx_hbm.at[pl.ds(c * _K, _K)], idx_v, sem_i)

    def drain_writes(s):
        (_idx, px_v, py_v, pz_v, _v, sem_w, _i) = s
        z = pl.ds(0, _K)
        pltpu.make_async_copy(px_v, ox_hbm.at[z], sem_w).wait()
        pltpu.make_async_copy(py_v, oy_hbm.at[z], sem_w).wait()
        pltpu.make_async_copy(pz_v, oz_hbm.at[z], sem_w).wait()

    def fire(j, s, drain):
        (idx_v, px_v, py_v, pz_v, sem_v, _w, sem_i) = s
        c = wid + _NW * j

        @pl.when(c < _NCHUNKS)
        def _():
            if drain:
                drain_writes(s)
            pltpu.make_async_copy(
                idx_hbm.at[pl.ds(c * _K, _K)], idx_v, sem_i).wait()
            pltpu.async_copy(vx_hbm.at[idx_v], px_v, sem_v)
            pltpu.async_copy(vy_hbm.at[idx_v], py_v, sem_v)
            pltpu.async_copy(vz_hbm.at[idx_v], pz_v, sem_v)

    def finish(j, s):
        (idx_v, px_v, py_v, pz_v, sem_v, sem_w, _i) = s
        c = wid + _NW * j

        @pl.when(c < _NCHUNKS)
        def _():
            base = c * _K
            pltpu.make_async_copy(vx_hbm.at[idx_v], px_v, sem_v).wait()
            pltpu.make_async_copy(vy_hbm.at[idx_v], py_v, sem_v).wait()
            pltpu.make_async_copy(vz_hbm.at[idx_v], pz_v, sem_v).wait()
            pltpu.async_copy(px_v, ox_hbm.at[pl.ds(base, _K)], sem_w)
            pltpu.async_copy(py_v, oy_hbm.at[pl.ds(base, _K)], sem_w)
            pltpu.async_copy(pz_v, oz_hbm.at[pl.ds(base, _K)], sem_w)
            prefetch_idx(j + 2, s)

    _sweep(prefetch_idx, fire, finish, drain_writes, sets)


@jax.jit
def _run(vec, distances, filter_indices):
    mesh = plsc.VectorSubcoreMesh(core_axis_name="c", subcore_axis_name="s")
    f32 = jnp.float32
    dist_scratch = [
        pltpu.VMEM((_K,), jnp.int32),
        pltpu.VMEM((_K,), f32),
        pltpu.VMEM((_K,), f32),
        pltpu.VMEM((_K,), jnp.int32),
    ]
    vec_scratch = [
        pltpu.VMEM((_K,), jnp.int32),
        pltpu.VMEM((_K,), f32),
        pltpu.VMEM((_K,), f32),
        pltpu.VMEM((_K,), f32),
    ]
    dist_fn = pl.kernel(
        _dist_body,
        out_type=[
            jax.ShapeDtypeStruct((_E_FILTER,), f32),
            jax.ShapeDtypeStruct((_E_FILTER,), f32),
            jax.ShapeDtypeStruct((_E_FILTER,), jnp.int32),
        ],
        mesh=mesh,
        scratch_types=dist_scratch * 2 + [pltpu.SemaphoreType.DMA] * 6,
    )
    vec_fn = pl.kernel(
        _vec_body,
        out_type=[
            jax.ShapeDtypeStruct((_E_FILTER,), f32),
            jax.ShapeDtypeStruct((_E_FILTER,), f32),
            jax.ShapeDtypeStruct((_E_FILTER,), f32),
        ],
        mesh=mesh,
        scratch_types=vec_scratch * 2 + [pltpu.SemaphoreType.DMA] * 6,
    )
    dist_f, switch, mask_i32 = dist_fn(distances, filter_indices)
    ox, oy, oz = vec_fn(vec[:, 0], vec[:, 1], vec[:, 2], filter_indices)
    vec_f = jnp.stack([ox, oy, oz], axis=1)
    return vec_f, dist_f, switch, mask_i32.astype(jnp.bool_)


def kernel(vec, distances, filter_indices):
    return _run(vec, distances, filter_indices)
